# Initial kernel scaffold; baseline (speedup 1.0000x reference)
import jax
import jax.numpy as jnp
from jax.experimental import pallas as pl


def kernel(x, edge_index, enc_w1, enc_b1, enc_w2, enc_b2, comm_w0, comm_b0, comm_w1, comm_b1, dec_w1, dec_b1, dec_w2, dec_b2):
    raise NotImplementedError("write your pallas kernel here")


# speedup vs baseline: 4.1028x; 4.1028x over previous
import functools

import jax
import jax.numpy as jnp
from jax import lax
from jax.experimental import pallas as pl
from jax.experimental.pallas import tpu as pltpu
from jax.experimental.pallas import tpu_sc as plsc

N = 10000
D = 128
H = 128
E0 = 320000

NC = 2
NS = 16
NW = NC * NS
NP = 10240
STRIPE = NP // NS
CH = 128
NCHUNK = 79
EPT = NCHUNK * CH
EP = EPT * NW


def _mesh():
    return plsc.VectorSubcoreMesh(
        core_axis_name="c", subcore_axis_name="s",
        num_cores=NC, num_subcores=NS)


@functools.cache
def _get_sc_counts():
    return functools.partial(
        pl.kernel,
        out_type=jax.ShapeDtypeStruct((NC * NP, 16), jnp.float32),
        mesh=_mesh(),
        scratch_types=[
            pltpu.VMEM((CH,), jnp.int32),
            pltpu.VMEM((CH, 16), jnp.float32),
            pltpu.VMEM_SHARED((NP, 16), jnp.float32),
        ],
    )(_sc_counts_body)


def _sc_counts_body(src_hbm, ones_hbm, zero_hbm, out_hbm, idx_c, ones_v, acc):
    cid = lax.axis_index("c")
    sid = lax.axis_index("s")
    wid = cid * NS + sid
    pltpu.sync_copy(ones_hbm, ones_v)
    pltpu.sync_copy(zero_hbm.at[pl.ds(sid * STRIPE, STRIPE)],
                    acc.at[pl.ds(sid * STRIPE, STRIPE)])
    plsc.subcore_barrier()

    def _body(j, _):
        pltpu.sync_copy(src_hbm.at[wid * NCHUNK + j], idx_c)
        pltpu.sync_copy(ones_v, acc.at[idx_c], add=True)
        return 0
    lax.fori_loop(0, NCHUNK, _body, 0)
    plsc.subcore_barrier()
    pltpu.sync_copy(acc.at[pl.ds(sid * STRIPE, STRIPE)],
                    out_hbm.at[pl.ds(cid * NP + sid * STRIPE, STRIPE)])


@functools.cache
def _get_sc_gather_scatter():
    return functools.partial(
        pl.kernel,
        out_type=jax.ShapeDtypeStruct((NC * NP, H), jnp.float32),
        mesh=_mesh(),
        scratch_types=[
            pltpu.VMEM((CH,), jnp.int32),
            pltpu.VMEM((CH,), jnp.int32),
            pltpu.VMEM((CH, H), jnp.float32),
            pltpu.VMEM_SHARED((NP, H), jnp.float32),
            pltpu.SemaphoreType.DMA,
        ],
    )(_sc_gs_body)


def _sc_gs_body(h_hbm, dst_hbm, src_hbm, zero_hbm, out_hbm,
                dst_c, src_c, rows, acc, sem):
    cid = lax.axis_index("c")
    sid = lax.axis_index("s")
    wid = cid * NS + sid
    pltpu.sync_copy(zero_hbm.at[pl.ds(sid * STRIPE, STRIPE)],
                    acc.at[pl.ds(sid * STRIPE, STRIPE)])
    plsc.subcore_barrier()

    def _body(j, _):
        pltpu.sync_copy(dst_hbm.at[wid * NCHUNK + j], dst_c)
        pltpu.sync_copy(src_hbm.at[wid * NCHUNK + j], src_c)
        pltpu.async_copy(h_hbm.at[dst_c], rows, sem).wait()
        pltpu.sync_copy(rows, acc.at[src_c], add=True)
        return 0
    lax.fori_loop(0, NCHUNK, _body, 0)
    plsc.subcore_barrier()
    pltpu.sync_copy(acc.at[pl.ds(sid * STRIPE, STRIPE)],
                    out_hbm.at[pl.ds(cid * NP + sid * STRIPE, STRIPE)])


_BR = 1024


def _mlp_body(x_ref, w1_ref, b1_ref, w2_ref, b2_ref, o_ref):
    t = jnp.dot(x_ref[...], w1_ref[...], preferred_element_type=jnp.float32)
    t = jnp.maximum(t + b1_ref[...], 0.0)
    o_ref[...] = (jnp.dot(t, w2_ref[...], preferred_element_type=jnp.float32)
                  + b2_ref[...])


def _tc_mlp(xp, w1, b1, w2, b2):
    return pl.pallas_call(
        _mlp_body,
        grid=(NP // _BR,),
        in_specs=[
            pl.BlockSpec((_BR, D), lambda i: (i, 0)),
            pl.BlockSpec((D, H), lambda i: (0, 0)),
            pl.BlockSpec((1, H), lambda i: (0, 0)),
            pl.BlockSpec((H, H), lambda i: (0, 0)),
            pl.BlockSpec((1, H), lambda i: (0, 0)),
        ],
        out_specs=pl.BlockSpec((_BR, H), lambda i: (i, 0)),
        out_shape=jax.ShapeDtypeStruct((NP, H), jnp.float32),
    )(xp, w1, b1.reshape(1, H), w2, b2.reshape(1, H))


def _comm_body(h_ref, p_ref, c_ref, w_ref, b_ref, o_ref):
    cnt = c_ref[0, :, 0:1] + c_ref[1, :, 0:1]
    sums = p_ref[0] + p_ref[1]
    msg = sums / jnp.maximum(cnt, 1.0)
    t = jnp.dot(msg, w_ref[...], preferred_element_type=jnp.float32)
    o_ref[...] = h_ref[...] + jnp.maximum(t + b_ref[...], 0.0)


def _tc_comm(h, p, c, w, b):
    return pl.pallas_call(
        _comm_body,
        grid=(NP // _BR,),
        in_specs=[
            pl.BlockSpec((_BR, H), lambda i: (i, 0)),
            pl.BlockSpec((NC, _BR, H), lambda i: (0, i, 0)),
            pl.BlockSpec((NC, _BR, 16), lambda i: (0, i, 0)),
            pl.BlockSpec((H, H), lambda i: (0, 0)),
            pl.BlockSpec((1, H), lambda i: (0, 0)),
        ],
        out_specs=pl.BlockSpec((_BR, H), lambda i: (i, 0)),
        out_shape=jax.ShapeDtypeStruct((NP, H), jnp.float32),
    )(h, p, c, w, b.reshape(1, H))


def kernel(x, edge_index, enc_w1, enc_b1, enc_w2, enc_b2,
           comm_w0, comm_b0, comm_w1, comm_b1,
           dec_w1, dec_b1, dec_w2, dec_b2):
    xp = jnp.zeros((NP, D), jnp.float32).at[:N].set(x)
    src = edge_index[0]
    dst = edge_index[1]
    pad = EP - E0
    srcp = jnp.concatenate(
        [src, jnp.full((pad,), N, jnp.int32)]).reshape(NW * NCHUNK, CH)
    dstp = jnp.concatenate(
        [dst, jnp.zeros((pad,), jnp.int32)]).reshape(NW * NCHUNK, CH)
    ones16 = jnp.ones((CH, 16), jnp.float32)
    zero16 = jnp.zeros((NP, 16), jnp.float32)
    zeroH = jnp.zeros((NP, H), jnp.float32)

    c = _get_sc_counts()(srcp, ones16, zero16).reshape(NC, NP, 16)
    h = _tc_mlp(xp, enc_w1, enc_b1, enc_w2, enc_b2)
    for (w, b) in ((comm_w0, comm_b0), (comm_w1, comm_b1)):
        p = _get_sc_gather_scatter()(h, dstp, srcp, zeroH).reshape(NC, NP, H)
        h = _tc_comm(h, p, c, w, b)
    out = _tc_mlp(h, dec_w1, dec_b1, dec_w2, dec_b2)
    return out[:N]
